# speedup vs baseline: 5.0365x; 5.0365x over previous
import jax
import jax.numpy as jnp
from jax.experimental import pallas as pl

_ROWS = 512


def _body(t_ref, o_ref):
    o_ref[...] = jnp.broadcast_to(t_ref[...][None], o_ref.shape)


def kernel(x, table):
    B, S = x.shape
    M, H = table.shape
    return pl.pallas_call(
        _body,
        grid=(S // _ROWS,),
        in_specs=[pl.BlockSpec((_ROWS, H), lambda j: (j, 0))],
        out_specs=pl.BlockSpec((B, _ROWS, H), lambda j: (0, j, 0)),
        out_shape=jax.ShapeDtypeStruct((B, S, H), table.dtype),
    )(table)
